# baseline (speedup 1.0000x reference)
import functools

import jax
import jax.numpy as jnp
from jax import lax
from jax.experimental import pallas as pl
from jax.experimental.pallas import tpu as pltpu
from jax.experimental.pallas import tpu_sc as plsc

_N_BINS = 8
_P = 1024
_D = 64
_G = 64
_T = 4608
_NSLICE = 1
_TS = _T // _NSLICE
_NW = 32
_TPW = _TS // _NW
_CW = 16
_NCHUNK = _TPW // _CW


def _match_kernel(edges_ref, x_ref, pat_ref, val_ref):
    xb = x_ref[...]
    pb = pat_ref[...]

    qx = jnp.zeros(xb.shape, jnp.float32)
    qp = jnp.zeros(pb.shape, jnp.float32)
    for i in range(7):
        e = edges_ref[i]
        qx = qx + (xb > e).astype(jnp.float32)
        qp = qp + (pb > e).astype(jnp.float32)

    acc = jnp.zeros((_P, xb.shape[0]), jnp.float32)
    for b in range(_N_BINS):
        a = (qx == b).astype(jnp.bfloat16)
        p1 = (qp == b).astype(jnp.bfloat16)
        acc = acc + lax.dot_general(
            p1, a, (((1,), (1,)), ((), ())),
            preferred_element_type=jnp.float32)

    counts = acc.astype(jnp.int32)
    rev = (_P - 1) - lax.broadcasted_iota(jnp.int32, acc.shape, 0)
    val = counts * _P + rev
    gmax = jnp.max(val.reshape(_G, 16, val.shape[1]), axis=1)
    val_ref[...] = gmax.T


def _chunk_reduce(buf, tmp, lane):
    m_all = jnp.full((16,), -1, jnp.int32)
    for t in range(_CW):
        m16 = buf[t, pl.ds(0, 16)]
        for g in range(1, _G // 16):
            m16 = jnp.maximum(m16, buf[t, pl.ds(g * 16, 16)])
        for sh in (8, 4, 2, 1):
            tmp[pl.ds(0, 16)] = m16
            tmp[pl.ds(16, 16)] = m16
            m16 = jnp.maximum(m16, tmp[pl.ds(sh, 16)])
        m_all = jnp.where(lane == t, m16, m_all)
    return m_all


def _sc_argmax(val_hbm, best_hbm, score_hbm, buf0, buf1, tmp, bb, sb,
               sem0, sem1):
    wid = lax.axis_index("s") * 2 + lax.axis_index("c")
    tbase = wid * _TPW
    lane = lax.iota(jnp.int32, 16)
    bufs = (buf0, buf1)
    sems = (sem0, sem1)

    cps = [None] * _NCHUNK
    cps[0] = pltpu.async_copy(
        val_hbm.at[pl.ds(tbase, _CW), :], bufs[0], sems[0])
    for c in range(_NCHUNK):
        cps[c].wait()
        if c + 1 < _NCHUNK:
            cps[c + 1] = pltpu.async_copy(
                val_hbm.at[pl.ds(tbase + (c + 1) * _CW, _CW), :],
                bufs[(c + 1) % 2], sems[(c + 1) % 2])
        m_all = _chunk_reduce(bufs[c % 2], tmp, lane)
        bb[pl.ds(c * _CW, _CW)] = (_P - 1) - (m_all & (_P - 1))
        sb[pl.ds(c * _CW, _CW)] = (m_all >> 10).astype(jnp.float32) * (1.0 / _D)
    pltpu.sync_copy(bb, best_hbm.at[pl.ds(tbase, _TPW)])
    pltpu.sync_copy(sb, score_hbm.at[pl.ds(tbase, _TPW)])


@functools.partial(
    pl.kernel,
    mesh=plsc.VectorSubcoreMesh(core_axis_name="c", subcore_axis_name="s"),
    out_type=[
        jax.ShapeDtypeStruct((_TS,), jnp.int32),
        jax.ShapeDtypeStruct((_TS,), jnp.float32),
    ],
    scratch_types=[
        pltpu.VMEM((_CW, _G), jnp.int32),
        pltpu.VMEM((_CW, _G), jnp.int32),
        pltpu.VMEM((32,), jnp.int32),
        pltpu.VMEM((_TPW,), jnp.int32),
        pltpu.VMEM((_TPW,), jnp.float32),
        pltpu.SemaphoreType.DMA,
        pltpu.SemaphoreType.DMA,
    ],
)
def _sc_argmax_call(val_hbm, best_hbm, score_hbm, buf0, buf1, tmp, bb, sb,
                    s0, s1):
    _sc_argmax(val_hbm, best_hbm, score_hbm, buf0, buf1, tmp, bb, sb, s0, s1)


def kernel(x, patterns, quantize_edges):
    B, S, D = x.shape
    t_tile = 512
    x2 = x.reshape(B * S, D)
    tc = pl.pallas_call(
        _match_kernel,
        grid=(_TS // t_tile,),
        in_specs=[
            pl.BlockSpec(memory_space=pltpu.SMEM),
            pl.BlockSpec((t_tile, D), lambda i: (i, 0)),
            pl.BlockSpec((_P, D), lambda i: (0, 0)),
        ],
        out_specs=pl.BlockSpec((t_tile, _G), lambda i: (i, 0)),
        out_shape=jax.ShapeDtypeStruct((_TS, _G), jnp.int32),
    )
    bests, scores = [], []
    for s in range(_NSLICE):
        val = tc(quantize_edges, lax.slice_in_dim(x2, s * _TS, (s + 1) * _TS),
                 patterns)
        b1, s1 = _sc_argmax_call(val)
        bests.append(b1)
        scores.append(s1)
    best = jnp.concatenate(bests)
    score = jnp.concatenate(scores)
    return best.reshape(B, S), score.reshape(B, S)


# speedup vs baseline: 1.0994x; 1.0994x over previous
import functools

import jax
import jax.numpy as jnp
from jax import lax
from jax.experimental import pallas as pl
from jax.experimental.pallas import tpu as pltpu
from jax.experimental.pallas import tpu_sc as plsc

_N_BINS = 8
_P = 1024
_D = 64
_G = 64
_T = 4608
_NSLICE = 3
_TS = _T // _NSLICE
_NW = 32
_TPW = _TS // _NW
_CW = 16
_NCHUNK = _TPW // _CW


def _match_kernel(edges_ref, x_ref, pat_ref, val_ref, poh_ref, rf_ref):
    @pl.when(pl.program_id(0) == 0)
    def _init():
        pb = pat_ref[...]
        qp = jnp.zeros(pb.shape, jnp.float32)
        for i in range(7):
            qp = qp + (pb > edges_ref[i]).astype(jnp.float32)
        for b in range(_N_BINS):
            poh_ref[b] = (qp == b).astype(jnp.bfloat16)
        iot = lax.broadcasted_iota(jnp.int32, (_P, 1), 0)
        rf_ref[...] = ((_P - 1) - iot).astype(jnp.float32) * (1.0 / _P)

    xb = x_ref[...]
    qx = jnp.zeros(xb.shape, jnp.float32)
    for i in range(7):
        qx = qx + (xb > edges_ref[i]).astype(jnp.float32)

    acc = jnp.zeros((_P, xb.shape[0]), jnp.float32)
    for b in range(_N_BINS):
        a = (qx == b).astype(jnp.bfloat16)
        acc = acc + lax.dot_general(
            poh_ref[b], a, (((1,), (1,)), ((), ())),
            preferred_element_type=jnp.float32)

    val = acc + rf_ref[...]
    gmax = jnp.max(val.reshape(_G, 16, val.shape[1]), axis=1)
    val_ref[...] = gmax.T


def _chunk_reduce(buf, tmp):
    m_all = jnp.full((16,), -1.0, jnp.float32)
    lane = lax.iota(jnp.int32, 16)
    for t in range(_CW):
        m16 = buf[t, pl.ds(0, 16)]
        for g in range(1, _G // 16):
            m16 = jnp.maximum(m16, buf[t, pl.ds(g * 16, 16)])
        for sh in (8, 4, 2, 1):
            tmp[pl.ds(0, 16)] = m16
            tmp[pl.ds(16, 16)] = m16
            m16 = jnp.maximum(m16, tmp[pl.ds(sh, 16)])
        m_all = jnp.where(lane == t, m16, m_all)
    return m_all


def _sc_argmax(val_hbm, best_hbm, score_hbm, buf0, buf1, tmp, bb, sb,
               sem0, sem1):
    wid = lax.axis_index("s") * 2 + lax.axis_index("c")
    tbase = wid * _TPW
    bufs = (buf0, buf1)
    sems = (sem0, sem1)

    cps = [None] * _NCHUNK
    cps[0] = pltpu.async_copy(
        val_hbm.at[pl.ds(tbase, _CW), :], bufs[0], sems[0])
    for c in range(_NCHUNK):
        cps[c].wait()
        if c + 1 < _NCHUNK:
            cps[c + 1] = pltpu.async_copy(
                val_hbm.at[pl.ds(tbase + (c + 1) * _CW, _CW), :],
                bufs[(c + 1) % 2], sems[(c + 1) % 2])
        m_all = _chunk_reduce(bufs[c % 2], tmp)
        cnt = m_all.astype(jnp.int32)
        cf = cnt.astype(jnp.float32)
        k = ((m_all - cf) * float(_P)).astype(jnp.int32)
        bb[pl.ds(c * _CW, _CW)] = (_P - 1) - k
        sb[pl.ds(c * _CW, _CW)] = cf * (1.0 / _D)
    pltpu.sync_copy(bb, best_hbm.at[pl.ds(tbase, _TPW)])
    pltpu.sync_copy(sb, score_hbm.at[pl.ds(tbase, _TPW)])


@functools.partial(
    pl.kernel,
    mesh=plsc.VectorSubcoreMesh(core_axis_name="c", subcore_axis_name="s"),
    out_type=[
        jax.ShapeDtypeStruct((_TS,), jnp.int32),
        jax.ShapeDtypeStruct((_TS,), jnp.float32),
    ],
    scratch_types=[
        pltpu.VMEM((_CW, _G), jnp.float32),
        pltpu.VMEM((_CW, _G), jnp.float32),
        pltpu.VMEM((32,), jnp.float32),
        pltpu.VMEM((_TPW,), jnp.int32),
        pltpu.VMEM((_TPW,), jnp.float32),
        pltpu.SemaphoreType.DMA,
        pltpu.SemaphoreType.DMA,
    ],
)
def _sc_argmax_call(val_hbm, best_hbm, score_hbm, buf0, buf1, tmp, bb, sb,
                    s0, s1):
    _sc_argmax(val_hbm, best_hbm, score_hbm, buf0, buf1, tmp, bb, sb, s0, s1)


def kernel(x, patterns, quantize_edges):
    B, S, D = x.shape
    t_tile = 512
    x2 = x.reshape(B * S, D)
    tc = pl.pallas_call(
        _match_kernel,
        grid=(_TS // t_tile,),
        in_specs=[
            pl.BlockSpec(memory_space=pltpu.SMEM),
            pl.BlockSpec((t_tile, D), lambda i: (i, 0)),
            pl.BlockSpec((_P, D), lambda i: (0, 0)),
        ],
        out_specs=pl.BlockSpec((t_tile, _G), lambda i: (i, 0)),
        out_shape=jax.ShapeDtypeStruct((_TS, _G), jnp.float32),
        scratch_shapes=[
            pltpu.VMEM((_N_BINS, _P, _D), jnp.bfloat16),
            pltpu.VMEM((_P, 1), jnp.float32),
        ],
    )
    bests, scores = [], []
    for s in range(_NSLICE):
        val = tc(quantize_edges, lax.slice_in_dim(x2, s * _TS, (s + 1) * _TS),
                 patterns)
        b1, s1 = _sc_argmax_call(val)
        bests.append(b1)
        scores.append(s1)
    best = jnp.concatenate(bests)
    score = jnp.concatenate(scores)
    return best.reshape(B, S), score.reshape(B, S)


# speedup vs baseline: 1.1041x; 1.0043x over previous
import functools

import jax
import jax.numpy as jnp
from jax import lax
from jax.experimental import pallas as pl
from jax.experimental.pallas import tpu as pltpu
from jax.experimental.pallas import tpu_sc as plsc

_N_BINS = 8
_P = 1024
_D = 64
_G = 64
_T = 4608
_NSLICE = 3
_TS = _T // _NSLICE
_NW = 32
_TPW = _TS // _NW
_CW = 16
_NCHUNK = _TPW // _CW


def _match_kernel(edges_ref, x_ref, pat_ref, val_ref, poh_ref, rf_ref):
    @pl.when(pl.program_id(0) == 0)
    def _init():
        pb = pat_ref[...]
        qp = jnp.zeros(pb.shape, jnp.float32)
        for i in range(7):
            qp = qp + (pb > edges_ref[i]).astype(jnp.float32)
        for b in range(_N_BINS):
            poh_ref[b] = (qp == b).astype(jnp.bfloat16)
        iot = lax.broadcasted_iota(jnp.int32, (_P, 1), 0)
        rf_ref[...] = ((_P - 1) - iot).astype(jnp.float32) * (1.0 / _P)

    xb = x_ref[...]
    qx = jnp.zeros(xb.shape, jnp.float32)
    for i in range(7):
        qx = qx + (xb > edges_ref[i]).astype(jnp.float32)

    acc = jnp.zeros((_P, xb.shape[0]), jnp.float32)
    for b in range(_N_BINS):
        a = (qx == b).astype(jnp.bfloat16)
        acc = acc + lax.dot_general(
            poh_ref[b], a, (((1,), (1,)), ((), ())),
            preferred_element_type=jnp.float32)

    val = acc + rf_ref[...]
    gmax = jnp.max(val.reshape(_G, 16, val.shape[1]), axis=1)
    val_ref[...] = gmax.T


def _chunk_reduce(buf, tmp):
    m_all = jnp.full((16,), -1.0, jnp.float32)
    lane = lax.iota(jnp.int32, 16)
    for t in range(_CW):
        m16 = buf[t, pl.ds(0, 16)]
        for g in range(1, _G // 16):
            m16 = jnp.maximum(m16, buf[t, pl.ds(g * 16, 16)])
        for sh in (8, 4, 2, 1):
            tmp[pl.ds(0, 16)] = m16
            tmp[pl.ds(16, 16)] = m16
            m16 = jnp.maximum(m16, tmp[pl.ds(sh, 16)])
        m_all = jnp.where(lane == t, m16, m_all)
    return m_all


def _sc_argmax(val_hbm, best_hbm, score_hbm, buf0, buf1, tmp, bb, sb,
               sem0, sem1):
    wid = lax.axis_index("s") * 2 + lax.axis_index("c")
    tbase = wid * _TPW
    bufs = (buf0, buf1)
    sems = (sem0, sem1)

    cps = [None] * _NCHUNK
    cps[0] = pltpu.async_copy(
        val_hbm.at[pl.ds(tbase, _CW), :], bufs[0], sems[0])
    for c in range(_NCHUNK):
        cps[c].wait()
        if c + 1 < _NCHUNK:
            cps[c + 1] = pltpu.async_copy(
                val_hbm.at[pl.ds(tbase + (c + 1) * _CW, _CW), :],
                bufs[(c + 1) % 2], sems[(c + 1) % 2])
        m_all = _chunk_reduce(bufs[c % 2], tmp)
        cnt = m_all.astype(jnp.int32)
        cf = cnt.astype(jnp.float32)
        k = ((m_all - cf) * float(_P)).astype(jnp.int32)
        bb[pl.ds(c * _CW, _CW)] = (_P - 1) - k
        sb[pl.ds(c * _CW, _CW)] = cf * (1.0 / _D)
    pltpu.sync_copy(bb, best_hbm.at[pl.ds(tbase, _TPW)])
    pltpu.sync_copy(sb, score_hbm.at[pl.ds(tbase, _TPW)])


@functools.partial(
    pl.kernel,
    mesh=plsc.VectorSubcoreMesh(core_axis_name="c", subcore_axis_name="s"),
    out_type=[
        jax.ShapeDtypeStruct((_TS,), jnp.int32),
        jax.ShapeDtypeStruct((_TS,), jnp.float32),
    ],
    scratch_types=[
        pltpu.VMEM((_CW, _G), jnp.float32),
        pltpu.VMEM((_CW, _G), jnp.float32),
        pltpu.VMEM((32,), jnp.float32),
        pltpu.VMEM((_TPW,), jnp.int32),
        pltpu.VMEM((_TPW,), jnp.float32),
        pltpu.SemaphoreType.DMA,
        pltpu.SemaphoreType.DMA,
    ],
)
def _sc_argmax_call(val_hbm, best_hbm, score_hbm, buf0, buf1, tmp, bb, sb,
                    s0, s1):
    _sc_argmax(val_hbm, best_hbm, score_hbm, buf0, buf1, tmp, bb, sb, s0, s1)


def kernel(x, patterns, quantize_edges):
    B, S, D = x.shape
    t_tile = 512
    x2 = x.reshape(B * S, D)
    n_steps = _TS // t_tile
    bests, scores = [], []
    for s in range(_NSLICE):
        val = pl.pallas_call(
            _match_kernel,
            grid=(n_steps,),
            in_specs=[
                pl.BlockSpec(memory_space=pltpu.SMEM),
                pl.BlockSpec((t_tile, D),
                             lambda i, s=s: (s * n_steps + i, 0)),
                pl.BlockSpec((_P, D), lambda i: (0, 0)),
            ],
            out_specs=pl.BlockSpec((t_tile, _G), lambda i: (i, 0)),
            out_shape=jax.ShapeDtypeStruct((_TS, _G), jnp.float32),
            scratch_shapes=[
                pltpu.VMEM((_N_BINS, _P, _D), jnp.bfloat16),
                pltpu.VMEM((_P, 1), jnp.float32),
            ],
        )(quantize_edges, x2, patterns)
        b1, s1 = _sc_argmax_call(val)
        bests.append(b1)
        scores.append(s1)
    best = jnp.concatenate(bests)
    score = jnp.concatenate(scores)
    return best.reshape(B, S), score.reshape(B, S)


# speedup vs baseline: 1.1088x; 1.0043x over previous
import functools

import jax
import jax.numpy as jnp
from jax import lax
from jax.experimental import pallas as pl
from jax.experimental.pallas import tpu as pltpu
from jax.experimental.pallas import tpu_sc as plsc

_N_BINS = 8
_P = 1024
_D = 64
_G = 64
_T = 4608
_NSLICE = 3
_TS = _T // _NSLICE
_NW = 32
_TPW = _TS // _NW
_CW = 16
_NCHUNK = _TPW // _CW


def _match_kernel(edges_ref, x_ref, pat_ref, val_ref, poh_ref, rf_ref):
    @pl.when(pl.program_id(0) == 0)
    def _init():
        pb = pat_ref[...]
        qp = jnp.zeros(pb.shape, jnp.float32)
        for i in range(7):
            qp = qp + (pb > edges_ref[i]).astype(jnp.float32)
        for b in range(_N_BINS):
            poh_ref[b] = (qp == b).astype(jnp.bfloat16)
        iot = lax.broadcasted_iota(jnp.int32, (_P, 1), 0)
        rf_ref[...] = ((_P - 1) - iot).astype(jnp.float32) * (1.0 / _P)

    xb = x_ref[...]
    qx = jnp.zeros(xb.shape, jnp.float32)
    for i in range(7):
        qx = qx + (xb > edges_ref[i]).astype(jnp.float32)

    acc = jnp.zeros((_P, xb.shape[0]), jnp.float32)
    for b in range(_N_BINS):
        a = (qx == b).astype(jnp.bfloat16)
        acc = acc + lax.dot_general(
            poh_ref[b], a, (((1,), (1,)), ((), ())),
            preferred_element_type=jnp.float32)

    val = acc + rf_ref[...]
    gmax = jnp.max(val.reshape(_G, 16, val.shape[1]), axis=1)
    val_ref[...] = gmax.T


def _sc_argmax(val_hbm, best_hbm, score_hbm, buf, tmp, bb, sb):
    wid = lax.axis_index("s") * 2 + lax.axis_index("c")
    tbase = wid * _TPW
    lane = lax.iota(jnp.int32, 16)

    def chunk_body(c, _):
        pltpu.sync_copy(val_hbm.at[pl.ds(tbase + c * _CW, _CW), :], buf)

        def tok_body(t, m_all):
            m16 = buf[t, pl.ds(0, 16)]
            for g in range(1, _G // 16):
                m16 = jnp.maximum(m16, buf[t, pl.ds(g * 16, 16)])
            for sh in (8, 4, 2, 1):
                tmp[pl.ds(0, 16)] = m16
                tmp[pl.ds(16, 16)] = m16
                m16 = jnp.maximum(m16, tmp[pl.ds(sh, 16)])
            return jnp.where(lane == t, m16, m_all)

        m_all = lax.fori_loop(0, _CW, tok_body,
                              jnp.full((16,), -1.0, jnp.float32))
        cnt = m_all.astype(jnp.int32)
        cf = cnt.astype(jnp.float32)
        k = ((m_all - cf) * float(_P)).astype(jnp.int32)
        bb[pl.ds(c * _CW, _CW)] = (_P - 1) - k
        sb[pl.ds(c * _CW, _CW)] = cf * (1.0 / _D)
        return 0

    lax.fori_loop(0, _NCHUNK, chunk_body, 0)
    pltpu.sync_copy(bb, best_hbm.at[pl.ds(tbase, _TPW)])
    pltpu.sync_copy(sb, score_hbm.at[pl.ds(tbase, _TPW)])


@functools.partial(
    pl.kernel,
    mesh=plsc.VectorSubcoreMesh(core_axis_name="c", subcore_axis_name="s"),
    out_type=[
        jax.ShapeDtypeStruct((_TS,), jnp.int32),
        jax.ShapeDtypeStruct((_TS,), jnp.float32),
    ],
    scratch_types=[
        pltpu.VMEM((_CW, _G), jnp.float32),
        pltpu.VMEM((32,), jnp.float32),
        pltpu.VMEM((_TPW,), jnp.int32),
        pltpu.VMEM((_TPW,), jnp.float32),
    ],
)
def _sc_argmax_call(val_hbm, best_hbm, score_hbm, buf, tmp, bb, sb):
    _sc_argmax(val_hbm, best_hbm, score_hbm, buf, tmp, bb, sb)


def kernel(x, patterns, quantize_edges):
    B, S, D = x.shape
    t_tile = 512
    x2 = x.reshape(B * S, D)
    n_steps = _TS // t_tile
    bests, scores = [], []
    for s in range(_NSLICE):
        val = pl.pallas_call(
            _match_kernel,
            grid=(n_steps,),
            in_specs=[
                pl.BlockSpec(memory_space=pltpu.SMEM),
                pl.BlockSpec((t_tile, D),
                             lambda i, s=s: (s * n_steps + i, 0)),
                pl.BlockSpec((_P, D), lambda i: (0, 0)),
            ],
            out_specs=pl.BlockSpec((t_tile, _G), lambda i: (i, 0)),
            out_shape=jax.ShapeDtypeStruct((_TS, _G), jnp.float32),
            scratch_shapes=[
                pltpu.VMEM((_N_BINS, _P, _D), jnp.bfloat16),
                pltpu.VMEM((_P, 1), jnp.float32),
            ],
        )(quantize_edges, x2, patterns)
        b1, s1 = _sc_argmax_call(val)
        bests.append(b1)
        scores.append(s1)
    best = jnp.concatenate(bests)
    score = jnp.concatenate(scores)
    return best.reshape(B, S), score.reshape(B, S)


# speedup vs baseline: 1.4197x; 1.2803x over previous
import functools

import jax
import jax.numpy as jnp
from jax import lax
from jax.experimental import pallas as pl
from jax.experimental.pallas import tpu as pltpu
from jax.experimental.pallas import tpu_sc as plsc

_N_BINS = 8
_P = 1024
_D = 64
_G = 64
_T = 4608
_SLICES = (2048, 2048, 512)
_NW = 32
_CW = 16
_TILE = 512


def _onehot_cat(v, edges_ref):
    gt = [(v > edges_ref[i]).astype(jnp.bfloat16) for i in range(7)]
    ohs = [1.0 - gt[0]]
    for b in range(1, 7):
        ohs.append(gt[b - 1] - gt[b])
    ohs.append(gt[6])
    return jnp.concatenate(ohs, axis=1)


def _match_kernel(edges_ref, x_ref, pat_ref, val_ref, poh_ref, rf_ref):
    @pl.when(pl.program_id(0) == 0)
    def _init():
        poh_ref[...] = _onehot_cat(pat_ref[...], edges_ref)
        iot = lax.broadcasted_iota(jnp.int32, (_P, 1), 0)
        rf_ref[...] = ((_P - 1) - iot).astype(jnp.float32) * (1.0 / _P)

    a_cat = _onehot_cat(x_ref[...], edges_ref)
    acc = lax.dot_general(
        poh_ref[...], a_cat, (((1,), (1,)), ((), ())),
        preferred_element_type=jnp.float32)

    val = acc + rf_ref[...]
    gmax = jnp.max(val.reshape(_G, 16, val.shape[1]), axis=1)
    val_ref[...] = gmax.T


def _sc_argmax(val_hbm, best_hbm, score_hbm, buf, tmp, bb, sb, *, tpw):
    wid = lax.axis_index("s") * 2 + lax.axis_index("c")
    tbase = wid * tpw
    lane = lax.iota(jnp.int32, 16)

    def chunk_body(c, _):
        pltpu.sync_copy(val_hbm.at[pl.ds(tbase + c * _CW, _CW), :], buf)

        def tok_body(t, m_all):
            m16 = buf[t, pl.ds(0, 16)]
            for g in range(1, _G // 16):
                m16 = jnp.maximum(m16, buf[t, pl.ds(g * 16, 16)])
            for sh in (8, 4, 2, 1):
                tmp[pl.ds(0, 16)] = m16
                tmp[pl.ds(16, 16)] = m16
                m16 = jnp.maximum(m16, tmp[pl.ds(sh, 16)])
            return jnp.where(lane == t, m16, m_all)

        m_all = lax.fori_loop(0, _CW, tok_body,
                              jnp.full((16,), -1.0, jnp.float32))
        cnt = m_all.astype(jnp.int32)
        cf = cnt.astype(jnp.float32)
        k = ((m_all - cf) * float(_P)).astype(jnp.int32)
        bb[pl.ds(c * _CW, _CW)] = (_P - 1) - k
        sb[pl.ds(c * _CW, _CW)] = cf * (1.0 / _D)
        return 0

    lax.fori_loop(0, tpw // _CW, chunk_body, 0)
    pltpu.sync_copy(bb, best_hbm.at[pl.ds(tbase, tpw)])
    pltpu.sync_copy(sb, score_hbm.at[pl.ds(tbase, tpw)])


@functools.lru_cache(maxsize=None)
def _make_sc_argmax(ts):
    tpw = ts // _NW

    @functools.partial(
        pl.kernel,
        mesh=plsc.VectorSubcoreMesh(core_axis_name="c", subcore_axis_name="s"),
        out_type=[
            jax.ShapeDtypeStruct((ts,), jnp.int32),
            jax.ShapeDtypeStruct((ts,), jnp.float32),
        ],
        scratch_types=[
            pltpu.VMEM((_CW, _G), jnp.float32),
            pltpu.VMEM((32,), jnp.float32),
            pltpu.VMEM((tpw,), jnp.int32),
            pltpu.VMEM((tpw,), jnp.float32),
        ],
    )
    def _call(val_hbm, best_hbm, score_hbm, buf, tmp, bb, sb):
        _sc_argmax(val_hbm, best_hbm, score_hbm, buf, tmp, bb, sb, tpw=tpw)

    return _call


def kernel(x, patterns, quantize_edges):
    B, S, D = x.shape
    x2 = x.reshape(B * S, D)
    bests, scores = [], []
    base = 0
    for ts in _SLICES:
        n_steps = ts // _TILE
        step0 = base // _TILE
        val = pl.pallas_call(
            _match_kernel,
            grid=(n_steps,),
            in_specs=[
                pl.BlockSpec(memory_space=pltpu.SMEM),
                pl.BlockSpec((_TILE, D),
                             lambda i, step0=step0: (step0 + i, 0)),
                pl.BlockSpec((_P, D), lambda i: (0, 0)),
            ],
            out_specs=pl.BlockSpec((_TILE, _G), lambda i: (i, 0)),
            out_shape=jax.ShapeDtypeStruct((ts, _G), jnp.float32),
            scratch_shapes=[
                pltpu.VMEM((_P, _N_BINS * _D), jnp.bfloat16),
                pltpu.VMEM((_P, 1), jnp.float32),
            ],
        )(quantize_edges, x2, patterns)
        b1, s1 = _make_sc_argmax(ts)(val)
        bests.append(b1)
        scores.append(s1)
        base += ts
    best = jnp.concatenate(bests)
    score = jnp.concatenate(scores)
    return best.reshape(B, S), score.reshape(B, S)
